# Initial kernel scaffold; baseline (speedup 1.0000x reference)
import jax
import jax.numpy as jnp
from jax.experimental import pallas as pl


def kernel(ego_psm, cav_psm, flag):
    raise NotImplementedError("write your pallas kernel here")


# speedup vs baseline: 14.3539x; 14.3539x over previous
import jax
import jax.numpy as jnp
from jax.experimental import pallas as pl
from jax.experimental.pallas import tpu as pltpu

_H = 768
_W = 768
_N = _H * _W
_K = max(1, int(_N * 0.3))
_CHUNK = 96
_NCH = _H // _CHUNK
_INT_MIN = -(2 ** 31)


def _mono_key(x):
    b = jax.lax.bitcast_convert_type(x, jnp.int32)
    return b ^ (jax.lax.shift_right_arithmetic(b, 31) & jnp.int32(0x7FFFFFFF))


def _key_to_float(m):
    b = m ^ (jax.lax.shift_right_arithmetic(m, 31) & jnp.int32(0x7FFFFFFF))
    return jax.lax.bitcast_convert_type(b, jnp.float32)


def _tc_body(ego_ref, cav_ref, mask_or_ref, mask_res_ref, mask_agg_ref,
             d_scr, s_scr, kd_scr, ks_scr):
    def prep_chunk(i, _):
        r = pl.ds(i * _CHUNK, _CHUNK)
        e0 = ego_ref[0, r, :]
        e1 = ego_ref[1, r, :]
        c0 = cav_ref[0, r, :]
        c1 = cav_ref[1, r, :]
        d = jnp.maximum(c0 - e0, c1 - e1)
        s = jnp.maximum(c0 + e0, c1 + e1)
        d_scr[r, :] = d
        s_scr[r, :] = s
        kd_scr[r, :] = _mono_key(d)
        ks_scr[r, :] = _mono_key(s)
        return 0

    jax.lax.fori_loop(0, _NCH, prep_chunk, 0)

    def count_ge(ref, cand):
        def chunk(i, acc):
            blk = ref[pl.ds(i * _CHUNK, _CHUNK), :]
            return acc + jnp.sum((blk >= cand).astype(jnp.int32))
        return jax.lax.fori_loop(0, _NCH, chunk, jnp.int32(0))

    zero = jnp.int32(0)
    imin = jnp.int32(_INT_MIN)
    md0 = jnp.where(count_ge(kd_scr, zero) >= _K, zero, imin)
    ms0 = jnp.where(count_ge(ks_scr, zero) >= _K, zero, imin)

    def bit_step(i, carry):
        md, ms = carry
        bit = jnp.left_shift(jnp.int32(1), 30 - i)
        cd = md | bit
        cs = ms | bit
        md = jnp.where(count_ge(kd_scr, cd) >= _K, cd, md)
        ms = jnp.where(count_ge(ks_scr, cs) >= _K, cs, ms)
        return md, ms

    md, ms = jax.lax.fori_loop(0, 31, bit_step, (md0, ms0))
    td = _key_to_float(md)
    ts = _key_to_float(ms)

    def mask_chunk(i, _):
        r = pl.ds(i * _CHUNK, _CHUNK)
        mr = (d_scr[r, :] >= td).astype(jnp.float32)
        ma = (s_scr[r, :] >= ts).astype(jnp.float32)
        mask_res_ref[0, r, :] = mr
        mask_agg_ref[0, r, :] = ma
        mask_or_ref[0, r, :] = jnp.maximum(mr, ma)
        return 0

    jax.lax.fori_loop(0, _NCH, mask_chunk, 0)


def kernel(ego_psm, cav_psm, flag):
    del flag
    out_sd = jax.ShapeDtypeStruct((1, _H, _W), jnp.float32)
    mask_or, mask_res, mask_agg = pl.pallas_call(
        _tc_body,
        out_shape=(out_sd, out_sd, out_sd),
        scratch_shapes=[
            pltpu.VMEM((_H, _W), jnp.float32),
            pltpu.VMEM((_H, _W), jnp.float32),
            pltpu.VMEM((_H, _W), jnp.int32),
            pltpu.VMEM((_H, _W), jnp.int32),
        ],
    )(ego_psm, cav_psm)
    return (mask_or, mask_res, mask_agg)
